# baseline (speedup 1.0000x reference)
import functools

import jax
import jax.numpy as jnp
from jax import lax
from jax.experimental import pallas as pl
from jax.experimental.pallas import tpu as pltpu
from jax.experimental.pallas import tpu_sc as plsc

_NUM_CORES = 2
_NUM_SUBCORES = 16
_NW = _NUM_CORES * _NUM_SUBCORES
_LANES = 16

_CHUNK = 128
_SUP_G = 5
_SUP = _SUP_G * _CHUNK

_C = 2


def _sc_gather_paired(embed, idx_c, pair):
    m = idx_c.shape[0]
    F = embed.shape[1]
    per_w = m // _NW
    n_sup = per_w // _SUP
    assert n_sup * _SUP == per_w
    n_blocks = per_w // (2 * pair)
    mesh = plsc.VectorSubcoreMesh(core_axis_name="c", subcore_axis_name="s")

    @functools.partial(
        pl.kernel,
        mesh=mesh,
        out_type=jax.ShapeDtypeStruct((m, F), jnp.float32),
        compiler_params=pltpu.CompilerParams(
            use_tc_tiling_on_sc=False, needs_layout_passes=False),
        scratch_types=[
            pltpu.VMEM((per_w,), jnp.int32),
            pltpu.VMEM((per_w,), jnp.int32),
            pltpu.VMEM((2, _SUP, F), jnp.float32),
            pltpu.SemaphoreType.DMA,
            pltpu.SemaphoreType.DMA,
            pltpu.SemaphoreType.DMA,
            pltpu.SemaphoreType.DMA,
        ],
    )
    def gather_kernel(table_hbm, idx_hbm, out_hbm, idx_v, int_v, rows_v,
                      gs0, gs1, ws0, ws1):
        wid = lax.axis_index("s") * _NUM_CORES + lax.axis_index("c")
        base = wid * per_w
        pltpu.sync_copy(idx_hbm.at[pl.ds(base, per_w)], idx_v)

        lane_pos = 2 * lax.iota(jnp.int32, _LANES)

        def ilv_body(i, carry):
            t = i // (pair // _LANES)
            k = i % (pair // _LANES)
            lo_off = t * 2 * pair + k * _LANES
            lo = jnp.maximum(idx_v[pl.ds(lo_off, _LANES)], 1)
            hi = jnp.maximum(idx_v[pl.ds(lo_off + pair, _LANES)], 1)
            pos = lane_pos + (t * 2 * pair + 2 * k * _LANES)
            plsc.store_scatter(int_v, [pos], lo)
            plsc.store_scatter(int_v, [pos + 1], hi)
            return carry

        lax.fori_loop(0, n_blocks * (pair // _LANES), ilv_body, 0, unroll=2)

        def do_super(s, slot, gsem, wsem):
            row0 = s * _SUP

            @pl.when(s >= 2)
            def _():
                pltpu.make_async_copy(
                    rows_v.at[slot], out_hbm.at[pl.ds(base, _SUP)], wsem
                ).wait()

            descs = [
                pltpu.async_copy(
                    table_hbm.at[int_v.at[pl.ds(row0 + j * _CHUNK, _CHUNK)]],
                    rows_v.at[slot, pl.ds(j * _CHUNK, _CHUNK)],
                    gsem,
                )
                for j in range(_SUP_G)
            ]
            for d in descs:
                d.wait()

            pltpu.async_copy(
                rows_v.at[slot], out_hbm.at[pl.ds(base + row0, _SUP)], wsem
            )

        def body(p, carry):
            do_super(2 * p, 0, gs0, ws0)
            do_super(2 * p + 1, 1, gs1, ws1)
            return carry

        lax.fori_loop(0, n_sup // 2, body, 0)
        if n_sup % 2:
            do_super(n_sup - 1, 0, gs0, ws0)
        pltpu.make_async_copy(rows_v.at[0], out_hbm.at[pl.ds(base, _SUP)], ws0).wait()
        pltpu.make_async_copy(rows_v.at[1], out_hbm.at[pl.ds(base, _SUP)], ws1).wait()

    return gather_kernel(embed, idx_c)


def _tc_film_chunk(e2_c, x2, w_pack, bg, bb, out_prev, start, pair):
    h_c, F2 = e2_c.shape
    M, D = x2.shape
    pb = 4
    blk2 = pb * pair
    blk = 2 * blk2
    nb_c = h_c // blk2
    assert start % blk == 0
    off = start // blk

    def body(e_ref, x_ref, w_ref, bg_ref, bb_ref, *rest):
        o_ref = rest[-1]
        one_bg = 1.0 + bg_ref[...]
        for t in range(pb):
            e_bf = e_ref[t * pair:(t + 1) * pair].astype(jnp.bfloat16)
            g = jnp.dot(e_bf, w_ref[...], preferred_element_type=jnp.float32)
            xlo = x_ref[2 * t * pair:(2 * t + 1) * pair]
            xhi = x_ref[(2 * t + 1) * pair:(2 * t + 2) * pair]
            o_ref[2 * t * pair:(2 * t + 1) * pair] = (
                g[:, :D] + one_bg) * xlo + (g[:, 2 * D:3 * D] + bb_ref[...])
            o_ref[(2 * t + 1) * pair:(2 * t + 2) * pair] = (
                g[:, D:2 * D] + one_bg) * xhi + (g[:, 3 * D:] + bb_ref[...])

    in_specs = [
        pl.BlockSpec((blk2, F2), lambda i: (i, 0)),
        pl.BlockSpec((blk, D), lambda i, _o=off: (i + _o, 0)),
        pl.BlockSpec((F2, 4 * D), lambda i: (0, 0)),
        pl.BlockSpec((1, D), lambda i: (0, 0)),
        pl.BlockSpec((1, D), lambda i: (0, 0)),
    ]
    args = [e2_c, x2, w_pack, bg, bb]
    aliases = {}
    if out_prev is not None:
        in_specs.append(pl.BlockSpec(memory_space=pl.ANY))
        args.append(out_prev)
        aliases = {5: 0}
    return pl.pallas_call(
        body,
        grid=(nb_c,),
        in_specs=in_specs,
        out_specs=pl.BlockSpec((blk, D), lambda i, _o=off: (i + _o, 0)),
        out_shape=jax.ShapeDtypeStruct((M, D), jnp.float32),
        input_output_aliases=aliases,
    )(*args)


def kernel(x, mods, embed, W_gamma, b_gamma, W_beta, b_beta):
    B, N, D = x.shape
    F = embed.shape[1]
    M = B * N
    sizes = [M // 5, 2 * M // 5, 2 * M // 5]
    idx = mods.reshape(M).astype(jnp.int32)
    x2 = x.reshape(M, D)
    zf = jnp.zeros((F, D), jnp.float32)
    w_pack = jnp.block([
        [W_gamma.T, zf, W_beta.T, zf],
        [zf, W_gamma.T, zf, W_beta.T],
    ]).astype(jnp.bfloat16)
    bg = b_gamma.reshape(1, D)
    bb = b_beta.reshape(1, D)

    e2s, starts, pairs = [], [], []
    pos = 0
    for m_c in sizes:
        pair_c = m_c // _NW // 2
        e_c = _sc_gather_paired(embed, lax.slice(idx, (pos,), (pos + m_c,)), pair_c)
        e2s.append(e_c.reshape(m_c // 2, 2 * F))
        starts.append(pos)
        pairs.append(pair_c)
        pos += m_c

    out = None
    for c in range(len(sizes)):
        out = _tc_film_chunk(e2s[c], x2, w_pack, bg, bb, out, starts[c], pairs[c])
    return out.reshape(B, N, D)


# speedup vs baseline: 1.0067x; 1.0067x over previous
import functools

import jax
import jax.numpy as jnp
from jax import lax
from jax.experimental import pallas as pl
from jax.experimental.pallas import tpu as pltpu
from jax.experimental.pallas import tpu_sc as plsc

_NUM_CORES = 2
_NUM_SUBCORES = 16
_NW = _NUM_CORES * _NUM_SUBCORES
_LANES = 16

_CHUNK = 128
_SUP_G = 5
_SUP = _SUP_G * _CHUNK

_C = 2


def _sc_gather_paired(embed, idx_c, pair):
    m = idx_c.shape[0]
    F = embed.shape[1]
    per_w = m // _NW
    n_sup = per_w // _SUP
    assert n_sup * _SUP == per_w
    n_blocks = per_w // (2 * pair)
    mesh = plsc.VectorSubcoreMesh(core_axis_name="c", subcore_axis_name="s")

    @functools.partial(
        pl.kernel,
        mesh=mesh,
        out_type=jax.ShapeDtypeStruct((m, F), jnp.float32),
        compiler_params=pltpu.CompilerParams(
            use_tc_tiling_on_sc=False, needs_layout_passes=False),
        scratch_types=[
            pltpu.VMEM((per_w,), jnp.int32),
            pltpu.VMEM((per_w,), jnp.int32),
            pltpu.VMEM((2, _SUP, F), jnp.float32),
            pltpu.SemaphoreType.DMA,
            pltpu.SemaphoreType.DMA,
            pltpu.SemaphoreType.DMA,
            pltpu.SemaphoreType.DMA,
        ],
    )
    def gather_kernel(table_hbm, idx_hbm, out_hbm, idx_v, int_v, rows_v,
                      gs0, gs1, ws0, ws1):
        wid = lax.axis_index("s") * _NUM_CORES + lax.axis_index("c")
        base = wid * per_w
        pltpu.sync_copy(idx_hbm.at[pl.ds(base, per_w)], idx_v)

        lane_pos = 2 * lax.iota(jnp.int32, _LANES)

        def ilv_body(i, carry):
            t = i // (pair // _LANES)
            k = i % (pair // _LANES)
            lo_off = t * 2 * pair + k * _LANES
            lo = jnp.maximum(idx_v[pl.ds(lo_off, _LANES)], 1)
            hi = jnp.maximum(idx_v[pl.ds(lo_off + pair, _LANES)], 1)
            pos = lane_pos + (t * 2 * pair + 2 * k * _LANES)
            plsc.store_scatter(int_v, [pos], lo)
            plsc.store_scatter(int_v, [pos + 1], hi)
            return carry

        lax.fori_loop(0, n_blocks * (pair // _LANES), ilv_body, 0, unroll=2)

        def do_super(s, slot, gsem, wsem):
            row0 = s * _SUP

            @pl.when(s >= 2)
            def _():
                pltpu.make_async_copy(
                    rows_v.at[slot], out_hbm.at[pl.ds(base, _SUP)], wsem
                ).wait()

            descs = [
                pltpu.async_copy(
                    table_hbm.at[int_v.at[pl.ds(row0 + j * _CHUNK, _CHUNK)]],
                    rows_v.at[slot, pl.ds(j * _CHUNK, _CHUNK)],
                    gsem,
                )
                for j in range(_SUP_G)
            ]
            for d in descs:
                d.wait()

            pltpu.async_copy(
                rows_v.at[slot], out_hbm.at[pl.ds(base + row0, _SUP)], wsem
            )

        def body(p, carry):
            do_super(2 * p, 0, gs0, ws0)
            do_super(2 * p + 1, 1, gs1, ws1)
            return carry

        lax.fori_loop(0, n_sup // 2, body, 0)
        if n_sup % 2:
            do_super(n_sup - 1, 0, gs0, ws0)
        pltpu.make_async_copy(rows_v.at[0], out_hbm.at[pl.ds(base, _SUP)], ws0).wait()
        pltpu.make_async_copy(rows_v.at[1], out_hbm.at[pl.ds(base, _SUP)], ws1).wait()

    return gather_kernel(embed, idx_c)


def _tc_film_chunk(e2_c, x2, w_pack, bg, bb, out_prev, c, pair):
    h_c, F2 = e2_c.shape
    M, D = x2.shape
    pb = 4
    blk2 = pb * pair
    blk = 2 * blk2
    nb_c = h_c // blk2
    off = c * nb_c

    def body(e_ref, x_ref, w_ref, bg_ref, bb_ref, *rest):
        o_ref = rest[-1]
        one_bg = 1.0 + bg_ref[...]
        for t in range(pb):
            e_bf = e_ref[t * pair:(t + 1) * pair].astype(jnp.bfloat16)
            g = jnp.dot(e_bf, w_ref[...], preferred_element_type=jnp.float32)
            xlo = x_ref[2 * t * pair:(2 * t + 1) * pair]
            xhi = x_ref[(2 * t + 1) * pair:(2 * t + 2) * pair]
            o_ref[2 * t * pair:(2 * t + 1) * pair] = (
                g[:, :D] + one_bg) * xlo + (g[:, 2 * D:3 * D] + bb_ref[...])
            o_ref[(2 * t + 1) * pair:(2 * t + 2) * pair] = (
                g[:, D:2 * D] + one_bg) * xhi + (g[:, 3 * D:] + bb_ref[...])

    in_specs = [
        pl.BlockSpec((blk2, F2), lambda i: (i, 0)),
        pl.BlockSpec((blk, D), lambda i, _o=off: (i + _o, 0)),
        pl.BlockSpec((F2, 4 * D), lambda i: (0, 0)),
        pl.BlockSpec((1, D), lambda i: (0, 0)),
        pl.BlockSpec((1, D), lambda i: (0, 0)),
    ]
    args = [e2_c, x2, w_pack, bg, bb]
    aliases = {}
    if out_prev is not None:
        in_specs.append(pl.BlockSpec(memory_space=pl.ANY))
        args.append(out_prev)
        aliases = {5: 0}
    return pl.pallas_call(
        body,
        grid=(nb_c,),
        in_specs=in_specs,
        out_specs=pl.BlockSpec((blk, D), lambda i, _o=off: (i + _o, 0)),
        out_shape=jax.ShapeDtypeStruct((M, D), jnp.float32),
        input_output_aliases=aliases,
    )(*args)


def kernel(x, mods, embed, W_gamma, b_gamma, W_beta, b_beta):
    B, N, D = x.shape
    F = embed.shape[1]
    M = B * N
    m_c = M // _C
    pair = m_c // _NW // 2
    idx = mods.reshape(M).astype(jnp.int32)
    x2 = x.reshape(M, D)
    zf = jnp.zeros((F, D), jnp.float32)
    w_pack = jnp.block([
        [W_gamma.T, zf, W_beta.T, zf],
        [zf, W_gamma.T, zf, W_beta.T],
    ]).astype(jnp.bfloat16)
    bg = b_gamma.reshape(1, D)
    bb = b_beta.reshape(1, D)

    e2s = []
    for c in range(_C):
        e_c = _sc_gather_paired(embed, lax.slice(idx, (c * m_c,), ((c + 1) * m_c,)), pair)
        e2s.append(e_c.reshape(m_c // 2, 2 * F))

    out = _tc_film_chunk(e2s[0], x2, w_pack, bg, bb, None, 0, pair)
    for c in range(1, _C):
        out = _tc_film_chunk(e2s[c], x2, w_pack, bg, bb, out, c, pair)
    return out.reshape(B, N, D)
